# baseline (speedup 1.0000x reference)
import jax
import jax.numpy as jnp
from jax.experimental import pallas as pl
from jax.experimental.pallas import tpu as pltpu

_EPS = 1e-4
_GT = 16


def _ltn_kernel(x_ref, wt_ref, b_ref, f_ref, y_ref, hid_ref,
                yscr, mu_ref, var_ref):
    s = pl.program_id(0)
    sblk, batch, hid = y_ref.shape
    groups = sblk // _GT
    rows = _GT * batch

    @pl.when(s == 0)
    def _():
        mu_ref[...] = -jnp.broadcast_to(b_ref[...], mu_ref.shape)
        var_ref[...] = jnp.ones_like(var_ref)

    wt = wt_ref[...]

    yscr[pl.ds(0, rows), :] = jnp.dot(
        x_ref[pl.ds(0, rows), :], wt,
        preferred_element_type=jnp.float32).astype(jnp.bfloat16)

    f = jnp.broadcast_to(f_ref[...], (batch, hid))
    mu = mu_ref[...]
    var = var_ref[...]

    for g in range(groups):
        for j in range(_GT):
            t = g * _GT + j
            y_t = yscr[pl.ds(t * batch, batch), :].astype(jnp.float32)
            diff = y_t - mu
            inv = jax.lax.rsqrt(var + _EPS)
            y_ref[t] = jnp.tanh(diff * inv)
            mu = mu + f * diff
            var = var + f * (diff * diff - var)
        dst = (g + 1) * rows
        src = min(g + 1, groups - 1) * rows
        yscr[pl.ds(dst, rows), :] = jnp.dot(
            x_ref[pl.ds(src, rows), :], wt,
            preferred_element_type=jnp.float32).astype(jnp.bfloat16)

    mu_ref[...] = mu
    var_ref[...] = var

    hid_ref[:, :hid] = mu + b_ref[...]
    hid_ref[:, hid:] = var


def kernel(x, weight, bias, forget_gate):
    S, B, H = x.shape
    SBLK = 128
    x2 = x.reshape(S * B, H)
    wt = weight.T
    b2 = bias.reshape(1, H)
    f2 = forget_gate.reshape(1, H)
    rows_blk = SBLK * B
    y_out, hid = pl.pallas_call(
        _ltn_kernel,
        grid=(S // SBLK,),
        in_specs=[
            pl.BlockSpec((rows_blk, H), lambda s: (s, 0)),
            pl.BlockSpec((H, H), lambda s: (0, 0)),
            pl.BlockSpec((1, H), lambda s: (0, 0)),
            pl.BlockSpec((1, H), lambda s: (0, 0)),
        ],
        out_specs=[
            pl.BlockSpec((SBLK, B, H), lambda s: (s, 0, 0)),
            pl.BlockSpec((B, 2 * H), lambda s: (0, 0)),
        ],
        out_shape=[
            jax.ShapeDtypeStruct((S, B, H), jnp.float32),
            jax.ShapeDtypeStruct((B, 2 * H), jnp.float32),
        ],
        scratch_shapes=[
            pltpu.VMEM((rows_blk + _GT * B, H), jnp.bfloat16),
            pltpu.VMEM((B, H), jnp.float32),
            pltpu.VMEM((B, H), jnp.float32),
        ],
        compiler_params=pltpu.CompilerParams(
            dimension_semantics=("arbitrary",),
            vmem_limit_bytes=52 * 1024 * 1024,
        ),
        name="linear_temp_norm",
    )(x2, wt, b2, f2)
    return y_out, hid


# speedup vs baseline: 1.0168x; 1.0168x over previous
import jax
import jax.numpy as jnp
from jax.experimental import pallas as pl
from jax.experimental.pallas import tpu as pltpu

_EPS = 1e-4
_GT = 16


def _ltn_kernel(x_ref, wt_ref, b_ref, f_ref, y_ref, hid_ref,
                yscr, mu_ref, var_ref):
    s = pl.program_id(0)
    sblk, batch, hid = y_ref.shape
    groups = sblk // _GT
    rows = _GT * batch

    @pl.when(s == 0)
    def _():
        mu_ref[...] = -jnp.broadcast_to(b_ref[...], mu_ref.shape)
        var_ref[...] = jnp.ones_like(var_ref)

    wt = wt_ref[...]

    yscr[pl.ds(0, rows), :] = jnp.dot(
        x_ref[pl.ds(0, rows), :], wt, preferred_element_type=jnp.float32)

    nh = 2
    hw = hid // nh
    f = [jnp.broadcast_to(f_ref[:, h * hw:(h + 1) * hw], (batch, hw))
         for h in range(nh)]
    mu = [mu_ref[:, h * hw:(h + 1) * hw] for h in range(nh)]
    var = [var_ref[:, h * hw:(h + 1) * hw] for h in range(nh)]

    for g in range(groups):
        for h in range(nh):
            cs = slice(h * hw, (h + 1) * hw)
            m, v, fh = mu[h], var[h], f[h]
            for j in range(_GT):
                t = g * _GT + j
                y_t = yscr[t * batch:(t + 1) * batch, cs]
                diff = y_t - m
                inv = jax.lax.rsqrt(v + _EPS)
                y_ref[t, :, cs] = jnp.tanh(diff * inv)
                m = m + fh * diff
                v = v + fh * (diff * diff - v)
            mu[h], var[h] = m, v
        dst = (g + 1) * rows
        src = min(g + 1, groups - 1) * rows
        yscr[pl.ds(dst, rows), :] = jnp.dot(
            x_ref[pl.ds(src, rows), :], wt, preferred_element_type=jnp.float32)

    for h in range(nh):
        cs = slice(h * hw, (h + 1) * hw)
        mu_ref[:, cs] = mu[h]
        var_ref[:, cs] = var[h]
        hid_ref[:, h * hw:(h + 1) * hw] = mu[h] + b_ref[:, cs]
        hid_ref[:, hid + h * hw:hid + (h + 1) * hw] = var[h]


def kernel(x, weight, bias, forget_gate):
    S, B, H = x.shape
    SBLK = 128
    x2 = x.reshape(S * B, H)
    wt = weight.T
    b2 = bias.reshape(1, H)
    f2 = forget_gate.reshape(1, H)
    rows_blk = SBLK * B
    y_out, hid = pl.pallas_call(
        _ltn_kernel,
        grid=(S // SBLK,),
        in_specs=[
            pl.BlockSpec((rows_blk, H), lambda s: (s, 0)),
            pl.BlockSpec((H, H), lambda s: (0, 0)),
            pl.BlockSpec((1, H), lambda s: (0, 0)),
            pl.BlockSpec((1, H), lambda s: (0, 0)),
        ],
        out_specs=[
            pl.BlockSpec((SBLK, B, H), lambda s: (s, 0, 0)),
            pl.BlockSpec((B, 2 * H), lambda s: (0, 0)),
        ],
        out_shape=[
            jax.ShapeDtypeStruct((S, B, H), jnp.float32),
            jax.ShapeDtypeStruct((B, 2 * H), jnp.float32),
        ],
        scratch_shapes=[
            pltpu.VMEM((rows_blk + _GT * B, H), jnp.float32),
            pltpu.VMEM((B, H), jnp.float32),
            pltpu.VMEM((B, H), jnp.float32),
        ],
        compiler_params=pltpu.CompilerParams(
            dimension_semantics=("arbitrary",),
            vmem_limit_bytes=52 * 1024 * 1024,
        ),
        name="linear_temp_norm",
    )(x2, wt, b2, f2)
    return y_out, hid


# speedup vs baseline: 1.0177x; 1.0010x over previous
import jax
import jax.numpy as jnp
from jax.experimental import pallas as pl
from jax.experimental.pallas import tpu as pltpu

_EPS = 1e-4
_GT = 16


def _ltn_kernel(x_ref, wt_ref, b_ref, f_ref, y_ref, hid_ref,
                yscr, mu_ref, var_ref):
    s = pl.program_id(0)
    sblk, batch, hid = y_ref.shape
    groups = sblk // _GT
    rows = _GT * batch

    @pl.when(s == 0)
    def _():
        mu_ref[...] = -jnp.broadcast_to(b_ref[...], mu_ref.shape)
        var_ref[...] = jnp.ones_like(var_ref)

    wt = wt_ref[...]

    yscr[pl.ds(0, rows), :] = jnp.dot(
        x_ref[pl.ds(0, rows), :], wt, preferred_element_type=jnp.float32)

    nh = 4
    hw = hid // nh
    f = [jnp.broadcast_to(f_ref[:, h * hw:(h + 1) * hw], (batch, hw))
         for h in range(nh)]
    mu = [mu_ref[:, h * hw:(h + 1) * hw] for h in range(nh)]
    var = [var_ref[:, h * hw:(h + 1) * hw] for h in range(nh)]

    for g in range(groups):
        for h in range(nh):
            cs = slice(h * hw, (h + 1) * hw)
            m, v, fh = mu[h], var[h], f[h]
            for j in range(_GT):
                t = g * _GT + j
                y_t = yscr[t * batch:(t + 1) * batch, cs]
                diff = y_t - m
                inv = jax.lax.rsqrt(v + _EPS)
                y_ref[t, :, cs] = jnp.tanh(diff * inv)
                m = m + fh * diff
                v = v + fh * (diff * diff - v)
            mu[h], var[h] = m, v
        dst = (g + 1) * rows
        src = min(g + 1, groups - 1) * rows
        yscr[pl.ds(dst, rows), :] = jnp.dot(
            x_ref[pl.ds(src, rows), :], wt, preferred_element_type=jnp.float32)

    for h in range(nh):
        cs = slice(h * hw, (h + 1) * hw)
        mu_ref[:, cs] = mu[h]
        var_ref[:, cs] = var[h]
        hid_ref[:, h * hw:(h + 1) * hw] = mu[h] + b_ref[:, cs]
        hid_ref[:, hid + h * hw:hid + (h + 1) * hw] = var[h]


def kernel(x, weight, bias, forget_gate):
    S, B, H = x.shape
    SBLK = 128
    x2 = x.reshape(S * B, H)
    wt = weight.T
    b2 = bias.reshape(1, H)
    f2 = forget_gate.reshape(1, H)
    rows_blk = SBLK * B
    y_out, hid = pl.pallas_call(
        _ltn_kernel,
        grid=(S // SBLK,),
        in_specs=[
            pl.BlockSpec((rows_blk, H), lambda s: (s, 0)),
            pl.BlockSpec((H, H), lambda s: (0, 0)),
            pl.BlockSpec((1, H), lambda s: (0, 0)),
            pl.BlockSpec((1, H), lambda s: (0, 0)),
        ],
        out_specs=[
            pl.BlockSpec((SBLK, B, H), lambda s: (s, 0, 0)),
            pl.BlockSpec((B, 2 * H), lambda s: (0, 0)),
        ],
        out_shape=[
            jax.ShapeDtypeStruct((S, B, H), jnp.float32),
            jax.ShapeDtypeStruct((B, 2 * H), jnp.float32),
        ],
        scratch_shapes=[
            pltpu.VMEM((rows_blk + _GT * B, H), jnp.float32),
            pltpu.VMEM((B, H), jnp.float32),
            pltpu.VMEM((B, H), jnp.float32),
        ],
        compiler_params=pltpu.CompilerParams(
            dimension_semantics=("arbitrary",),
            vmem_limit_bytes=52 * 1024 * 1024,
        ),
        name="linear_temp_norm",
    )(x2, wt, b2, f2)
    return y_out, hid


# speedup vs baseline: 1.0409x; 1.0227x over previous
import jax
import jax.numpy as jnp
from jax.experimental import pallas as pl
from jax.experimental.pallas import tpu as pltpu

_EPS = 1e-4
_GT = 16


def _ltn_kernel(x_ref, wt_ref, b_ref, f_ref, y_ref, hid_ref,
                yscr, mu_ref, var_ref):
    s = pl.program_id(0)
    sblk, batch, hid = y_ref.shape
    groups = sblk // _GT
    rows = _GT * batch

    @pl.when(s == 0)
    def _():
        mu_ref[...] = -jnp.broadcast_to(b_ref[...], mu_ref.shape)
        var_ref[...] = jnp.ones_like(var_ref)

    wt = wt_ref[...]

    yscr[pl.ds(0, rows), :] = jnp.dot(
        x_ref[pl.ds(0, rows), :], wt, preferred_element_type=jnp.float32)

    nh = 4
    hw = hid // nh
    f = [jnp.broadcast_to(f_ref[:, h * hw:(h + 1) * hw], (batch, hw))
         for h in range(nh)]
    mu = [mu_ref[:, h * hw:(h + 1) * hw] for h in range(nh)]
    var = [var_ref[:, h * hw:(h + 1) * hw] for h in range(nh)]

    for g in range(groups):
        for h in range(nh):
            cs = slice(h * hw, (h + 1) * hw)
            m, v, fh = mu[h], var[h], f[h]
            for j in range(_GT):
                t = g * _GT + j
                y_t = yscr[t * batch:(t + 1) * batch, cs]
                diff = y_t - m
                inv = jax.lax.rsqrt(v + _EPS)
                y_ref[t, :, cs] = jnp.tanh(diff * inv)
                m = m + fh * diff
                v = v + fh * (diff * diff - v)
            mu[h], var[h] = m, v
        if g + 1 < groups:
            dst = (g + 1) * rows
            yscr[pl.ds(dst, rows), :] = jnp.dot(
                x_ref[pl.ds(dst, rows), :], wt,
                preferred_element_type=jnp.float32)

    for h in range(nh):
        cs = slice(h * hw, (h + 1) * hw)
        mu_ref[:, cs] = mu[h]
        var_ref[:, cs] = var[h]
        hid_ref[:, h * hw:(h + 1) * hw] = mu[h] + b_ref[:, cs]
        hid_ref[:, hid + h * hw:hid + (h + 1) * hw] = var[h]


def kernel(x, weight, bias, forget_gate):
    S, B, H = x.shape
    SBLK = 128
    x2 = x.reshape(S * B, H)
    wt = weight.T
    b2 = bias.reshape(1, H)
    f2 = forget_gate.reshape(1, H)
    rows_blk = SBLK * B
    y_out, hid = pl.pallas_call(
        _ltn_kernel,
        grid=(S // SBLK,),
        in_specs=[
            pl.BlockSpec((rows_blk, H), lambda s: (s, 0)),
            pl.BlockSpec((H, H), lambda s: (0, 0)),
            pl.BlockSpec((1, H), lambda s: (0, 0)),
            pl.BlockSpec((1, H), lambda s: (0, 0)),
        ],
        out_specs=[
            pl.BlockSpec((SBLK, B, H), lambda s: (s, 0, 0)),
            pl.BlockSpec((B, 2 * H), lambda s: (0, 0)),
        ],
        out_shape=[
            jax.ShapeDtypeStruct((S, B, H), jnp.float32),
            jax.ShapeDtypeStruct((B, 2 * H), jnp.float32),
        ],
        scratch_shapes=[
            pltpu.VMEM((rows_blk, H), jnp.float32),
            pltpu.VMEM((B, H), jnp.float32),
            pltpu.VMEM((B, H), jnp.float32),
        ],
        compiler_params=pltpu.CompilerParams(
            dimension_semantics=("arbitrary",),
            vmem_limit_bytes=52 * 1024 * 1024,
        ),
        name="linear_temp_norm",
    )(x2, wt, b2, f2)
    return y_out, hid
